# Initial kernel scaffold; baseline (speedup 1.0000x reference)
import jax
import jax.numpy as jnp
from jax.experimental import pallas as pl


def kernel(x, edge_index, W1, b1, g1, be1, W2, b2, g2, be2, PW1, Pb1, PW2, Pb2):
    raise NotImplementedError("write your pallas kernel here")


# speedup vs baseline: 15.6246x; 15.6246x over previous
import functools
import numpy as np
import jax
import jax.numpy as jnp
from jax import lax
from jax.experimental import pallas as pl
from jax.experimental.pallas import tpu as pltpu
from jax.experimental.pallas import tpu_sc as plsc

NC = 2
NS = 16
NW = NC * NS
L = 16

K = 125
IDX_BLK = 16
ZR = 64
BN_INV = float(1.0 / np.sqrt(1.0 + 1e-5))


def _flat_id():
    return lax.axis_index("c") * NS + lax.axis_index("s")


def _zero_vmem_rows(ref, rows, cols):
    def row(i, _):
        for k in range(cols // L):
            ref[i, pl.ds(k * L, L)] = jnp.zeros((L,), jnp.float32)
        return 0

    lax.fori_loop(0, rows, row, 0)


def _agg_body(n_pad, n_chunks, hs_hbm, src_hbm, dst_hbm, out_hbm,
              src_v, dst_v, rows_v, zrow_v, acc, sem):
    c = lax.axis_index("c")
    s = lax.axis_index("s")
    wid = _flat_id()
    d = hs_hbm.shape[1]
    rpt = n_pad // NS

    _zero_vmem_rows(zrow_v, ZR, d)
    for t in range(rpt // ZR):
        pltpu.sync_copy(zrow_v, acc.at[pl.ds(s * rpt + t * ZR, ZR)])
    plsc.subcore_barrier()

    n_blk = src_v.shape[0]

    def block(b, _):
        pltpu.sync_copy(src_hbm.at[wid, pl.ds(b * n_blk, n_blk)], src_v)
        pltpu.sync_copy(dst_hbm.at[wid, pl.ds(b * n_blk, n_blk)], dst_v)

        def chunk(j, _):
            pltpu.async_copy(hs_hbm.at[src_v.at[j]], rows_v, sem).wait()
            pltpu.sync_copy(rows_v, acc.at[dst_v.at[j]], add=True)
            return 0

        lax.fori_loop(0, n_blk, chunk, 0)
        return 0

    lax.fori_loop(0, n_chunks // n_blk, block, 0)
    plsc.subcore_barrier()

    for t in range(rpt // ZR):
        r0 = s * rpt + t * ZR
        pltpu.sync_copy(acc.at[pl.ds(r0, ZR)], out_hbm.at[c, pl.ds(r0, ZR)])


def _sc_mesh():
    return plsc.VectorSubcoreMesh(core_axis_name="c", subcore_axis_name="s")


def _aggregate(hs, src3, dst3, n_pad, n_chunks):
    d = hs.shape[1]
    return pl.kernel(
        functools.partial(_agg_body, n_pad, n_chunks),
        out_type=jax.ShapeDtypeStruct((NC, n_pad, d), jnp.float32),
        mesh=_sc_mesh(),
        scratch_types=[
            pltpu.VMEM((IDX_BLK, K), jnp.int32),
            pltpu.VMEM((IDX_BLK, K), jnp.int32),
            pltpu.VMEM((K, d), jnp.float32),
            pltpu.VMEM((ZR, d), jnp.float32),
            pltpu.VMEM_SHARED((n_pad, d), jnp.float32),
            pltpu.SemaphoreType.DMA,
        ],
        name="gcn_aggregate_sc",
    )(hs, src3, dst3)


def _stage1_body(dega, degb, xb, w1, dis_o, hs_o):
    deg = dega[...] + degb[...] + 1.0
    dis = lax.rsqrt(deg)
    h = jnp.dot(xb[...], w1[...], preferred_element_type=jnp.float32)
    dis_o[...] = dis
    hs_o[...] = h * dis


def _stage2_body(a0, a1, hs1, dis, b1, g1, be1, w2, hs2_o):
    pre = dis[...] * (a0[...] + a1[...] + hs1[...]) + b1[...]
    bn = pre * (g1[...] * BN_INV) + be1[...]
    r = jnp.maximum(bn, 0.0)
    h2 = jnp.dot(r, w2[...], preferred_element_type=jnp.float32)
    hs2_o[...] = h2 * dis[...]


def _stage3_body(a0, a1, hs2, dis, b2, g2, be2, pw1, pb1, pw2, pb2, h_o, z_o):
    pre = dis[...] * (a0[...] + a1[...] + hs2[...]) + b2[...]
    h = pre * (g2[...] * BN_INV) + be2[...]
    h_o[...] = h
    t = jnp.maximum(
        jnp.dot(h, pw1[...], preferred_element_type=jnp.float32) + pb1[...], 0.0
    )
    z_o[...] = jnp.dot(t, pw2[...], preferred_element_type=jnp.float32) + pb2[...]


def _row_spec(r, d):
    return pl.BlockSpec((r, d), lambda i: (i, 0))


def _rep_spec(shape):
    return pl.BlockSpec(shape, lambda i: tuple(0 for _ in shape))


def kernel(x, edge_index, W1, b1, g1, be1, W2, b2, g2, be2, PW1, Pb1, PW2, Pb2):
    n, d = x.shape
    e = edge_index.shape[1]
    e_per = e // NW
    assert e % NW == 0 and e_per % K == 0
    n_chunks = e_per // K
    n_pad = ((n + NS * ZR - 1) // (NS * ZR)) * (NS * ZR)
    r = n_pad // 8
    grid = n_pad // r

    src3 = edge_index[0].reshape(NW, n_chunks, K)
    dst3 = edge_index[1].reshape(NW, n_chunks, K)
    xp = jnp.pad(x, ((0, n_pad - n), (0, 0)))

    deg2 = _aggregate(jnp.ones((n_pad, d), jnp.float32), src3, dst3, n_pad, n_chunks)
    dega = deg2[0, :, :1]
    degb = deg2[1, :, :1]

    vecs = [v.reshape(1, d) for v in (b1, g1, be1, b2, g2, be2, Pb1, Pb2)]
    b1r, g1r, be1r, b2r, g2r, be2r, pb1r, pb2r = vecs

    dis, hs1 = pl.pallas_call(
        _stage1_body,
        grid=(grid,),
        in_specs=[_row_spec(r, 1), _row_spec(r, 1), _row_spec(r, d), _rep_spec((d, d))],
        out_specs=[_row_spec(r, 1), _row_spec(r, d)],
        out_shape=[
            jax.ShapeDtypeStruct((n_pad, 1), jnp.float32),
            jax.ShapeDtypeStruct((n_pad, d), jnp.float32),
        ],
        name="gcn_stage1_tc",
    )(dega, degb, xp, W1)

    agg1 = _aggregate(hs1, src3, dst3, n_pad, n_chunks)

    hs2 = pl.pallas_call(
        _stage2_body,
        grid=(grid,),
        in_specs=[
            _row_spec(r, d), _row_spec(r, d), _row_spec(r, d), _row_spec(r, 1),
            _rep_spec((1, d)), _rep_spec((1, d)), _rep_spec((1, d)),
            _rep_spec((d, d)),
        ],
        out_specs=[_row_spec(r, d)],
        out_shape=[jax.ShapeDtypeStruct((n_pad, d), jnp.float32)],
        name="gcn_stage2_tc",
    )(agg1[0], agg1[1], hs1, dis, b1r, g1r, be1r, W2)[0]

    agg2 = _aggregate(hs2, src3, dst3, n_pad, n_chunks)

    h, z = pl.pallas_call(
        _stage3_body,
        grid=(grid,),
        in_specs=[
            _row_spec(r, d), _row_spec(r, d), _row_spec(r, d), _row_spec(r, 1),
            _rep_spec((1, d)), _rep_spec((1, d)), _rep_spec((1, d)),
            _rep_spec((d, d)), _rep_spec((1, d)), _rep_spec((d, d)),
            _rep_spec((1, d)),
        ],
        out_specs=[_row_spec(r, d), _row_spec(r, d)],
        out_shape=[
            jax.ShapeDtypeStruct((n_pad, d), jnp.float32),
            jax.ShapeDtypeStruct((n_pad, d), jnp.float32),
        ],
        name="gcn_stage3_tc",
    )(agg2[0], agg2[1], hs2, dis, b2r, g2r, be2r, PW1, pb1r, PW2, pb2r)

    return (h[:n], z[:n])
